# baseline (speedup 1.0000x reference)
import functools
import math

import jax
import jax.numpy as jnp
from jax import lax
from jax.experimental import pallas as pl
from jax.experimental.pallas import tpu as pltpu
from jax.experimental.pallas import tpu_sc as plsc

B = 4
N = 8192
NPOINT = 1024
NS_MAX = 32
EPS = 1e-5
BLK = 64


def _select_body(xyz_ref, c_ref, idx_ref):
    px = xyz_ref[0, 0, :][None, :]
    py = xyz_ref[0, 1, :][None, :]
    pz = xyz_ref[0, 2, :][None, :]
    cx = c_ref[0, :, 0:1]
    cy = c_ref[0, :, 1:2]
    cz = c_ref[0, :, 2:3]
    dx = cx - px
    dy = cy - py
    dz = cz - pz
    d2 = (dx * dx + dy * dy) + dz * dz
    lane = lax.broadcasted_iota(jnp.int32, (BLK, N), 1)
    big = jnp.float32(jnp.inf)
    idx_cols = []
    for _ in range(NS_MAX // 4):
        j0 = jnp.argmin(d2, axis=1, keepdims=True)
        j1 = jnp.argmin(jnp.where(lane == j0, big, d2), axis=1, keepdims=True)
        j2 = jnp.argmin(
            jnp.where((lane == j0) | (lane == j1), big, d2),
            axis=1, keepdims=True)
        j3 = jnp.argmin(
            jnp.where((lane == j0) | (lane == j1) | (lane == j2), big, d2),
            axis=1, keepdims=True)
        d2 = jnp.where(
            (lane == j0) | (lane == j1) | (lane == j2) | (lane == j3), big, d2)
        idx_cols += [j.astype(jnp.int32) for j in (j0, j1, j2, j3)]
    idx_ref[0] = jnp.concatenate(idx_cols, axis=1)


def _select(xyz_t, new_xyz):
    grid = (B, NPOINT // BLK)
    return pl.pallas_call(
        _select_body,
        grid=grid,
        in_specs=[
            pl.BlockSpec((1, 3, N), lambda b, i: (b, 0, 0)),
            pl.BlockSpec((1, BLK, 3), lambda b, i: (b, i, 0)),
        ],
        out_specs=pl.BlockSpec((1, BLK, NS_MAX), lambda b, i: (b, i, 0)),
        out_shape=jax.ShapeDtypeStruct((B, NPOINT, NS_MAX), jnp.int32),
    )(xyz_t, new_xyz)


_TOTAL = B * NPOINT * NS_MAX
_NW = 32
_CHUNK = _TOTAL // _NW


def _gather_body(idx_hbm, xyz_hbm, ox_hbm, oy_hbm, oz_hbm,
                 idx_v, xs_v, ys_v, zs_v, ox_v, oy_v, oz_v):
    wid = lax.axis_index("s") * 2 + lax.axis_index("c")
    batch = wid // (_NW // B)
    base = wid * _CHUNK
    pltpu.sync_copy(idx_hbm.at[pl.ds(base, _CHUNK)], idx_v)
    pltpu.sync_copy(xyz_hbm.at[batch * 3 + 0], xs_v)
    pltpu.sync_copy(xyz_hbm.at[batch * 3 + 1], ys_v)
    pltpu.sync_copy(xyz_hbm.at[batch * 3 + 2], zs_v)
    for i in range(_CHUNK // 16):
        iv = idx_v[pl.ds(i * 16, 16)]
        ox_v[pl.ds(i * 16, 16)] = plsc.load_gather(xs_v, [iv])
        oy_v[pl.ds(i * 16, 16)] = plsc.load_gather(ys_v, [iv])
        oz_v[pl.ds(i * 16, 16)] = plsc.load_gather(zs_v, [iv])
    pltpu.sync_copy(ox_v, ox_hbm.at[pl.ds(base, _CHUNK)])
    pltpu.sync_copy(oy_v, oy_hbm.at[pl.ds(base, _CHUNK)])
    pltpu.sync_copy(oz_v, oz_hbm.at[pl.ds(base, _CHUNK)])


def _gather(idx_flat, xyz_rows):
    mesh = plsc.VectorSubcoreMesh(core_axis_name="c", subcore_axis_name="s")
    k = functools.partial(
        pl.kernel,
        mesh=mesh,
        compiler_params=pltpu.CompilerParams(needs_layout_passes=False),
        out_type=[jax.ShapeDtypeStruct((_TOTAL,), jnp.float32)] * 3,
        scratch_types=[
            pltpu.VMEM((_CHUNK,), jnp.int32),
            pltpu.VMEM((N,), jnp.float32),
            pltpu.VMEM((N,), jnp.float32),
            pltpu.VMEM((N,), jnp.float32),
            pltpu.VMEM((_CHUNK,), jnp.float32),
            pltpu.VMEM((_CHUNK,), jnp.float32),
            pltpu.VMEM((_CHUNK,), jnp.float32),
        ],
    )(_gather_body)
    return k(idx_flat, xyz_rows)


def _branch_body(r2, inv_sqrt_d,
                 gx_ref, gy_ref, gz_ref, c_ref,
                 w1x_ref, w1y_ref, w1z_ref, b1_ref,
                 w2_ref, b2_ref, w3_ref, b3_ref,
                 wq_ref, wk_ref, wv_ref, out_ref):
    dot = functools.partial(
        lax.dot_general,
        preferred_element_type=jnp.float32,
    )
    mm = lambda a, b: dot(a, b, (((1,), (0,)), ((), ())))

    cx = c_ref[0, :, 0:1]
    cy = c_ref[0, :, 1:2]
    cz = c_ref[0, :, 2:3]
    relx = gx_ref[0] - cx
    rely = gy_ref[0] - cy
    relz = gz_ref[0] - cz
    d2 = (relx * relx + rely * rely) + relz * relz
    m = d2 <= r2
    relx = jnp.where(m, relx, jnp.broadcast_to(relx[:, 0:1], relx.shape))
    rely = jnp.where(m, rely, jnp.broadcast_to(rely[:, 0:1], rely.shape))
    relz = jnp.where(m, relz, jnp.broadcast_to(relz[:, 0:1], relz.shape))

    h = mm(relx, w1x_ref[...]) + mm(rely, w1y_ref[...]) + mm(relz, w1z_ref[...])
    h = jnp.maximum(h + b1_ref[...], 0.0)
    h = jnp.maximum(mm(h, w2_ref[...]) + b2_ref[...], 0.0)
    h = jnp.maximum(mm(h, w3_ref[...]) + b3_ref[...], 0.0)
    q = mm(h, wq_ref[...])
    k = mm(h, wk_ref[...])
    v = mm(h, wv_ref[...])
    s = dot(q, k, (((1,), (1,)), ((), ()))) * inv_sqrt_d
    smax = jnp.max(s, axis=1, keepdims=True)
    e = jnp.exp(s - smax)
    a = e / jnp.sum(e, axis=1, keepdims=True)
    out_ref[0] = mm(a, v)


def _fold_layer(layer):
    s = (layer["g1"] * layer["g2"]) / (1.0 + EPS)
    return layer["W"] * s[None, :], (layer["b"] * s)[None, :]


def _branch(gx, gy, gz, new_xyz, params, radius, ns, d1):
    w1, b1 = _fold_layer(params["layers"][0])
    w2, b2 = _fold_layer(params["layers"][1])
    w3, b3 = _fold_layer(params["layers"][2])
    w1x, w1y, w1z = w1[0::3], w1[1::3], w1[2::3]
    body = functools.partial(_branch_body, radius * radius, 1.0 / math.sqrt(128.0))
    full = lambda a: pl.BlockSpec(a.shape, lambda b: (0,) * a.ndim)
    args = (gx, gy, gz, new_xyz,
            w1x, w1y, w1z, b1, w2, b2, w3, b3,
            params["Wq"], params["Wk"], params["Wv"])
    in_specs = [
        pl.BlockSpec((1, NPOINT, ns), lambda b: (b, 0, 0)),
        pl.BlockSpec((1, NPOINT, ns), lambda b: (b, 0, 0)),
        pl.BlockSpec((1, NPOINT, ns), lambda b: (b, 0, 0)),
        pl.BlockSpec((1, NPOINT, 3), lambda b: (b, 0, 0)),
    ] + [full(a) for a in args[4:]]
    return pl.pallas_call(
        body,
        grid=(B,),
        in_specs=in_specs,
        out_specs=pl.BlockSpec((1, NPOINT, 128), lambda b: (b, 0, 0)),
        out_shape=jax.ShapeDtypeStruct((B, NPOINT, 128), jnp.float32),
    )(*args)


def kernel(xyz, params0, params1):
    new_xyz = xyz[:, :NPOINT, :]
    xyz_t = jnp.transpose(xyz, (0, 2, 1))
    idx = _select(xyz_t, new_xyz)
    gx, gy, gz = _gather(idx.reshape(-1), xyz_t.reshape(B * 3, N))
    gx = gx.reshape(B, NPOINT, NS_MAX)
    gy = gy.reshape(B, NPOINT, NS_MAX)
    gz = gz.reshape(B, NPOINT, NS_MAX)
    outs = []
    for params, radius, ns, d1 in ((params0, 0.1, 16, 64), (params1, 0.2, 32, 96)):
        outs.append(_branch(gx[..., :ns], gy[..., :ns], gz[..., :ns],
                            new_xyz, params, radius, ns, d1))
    return (new_xyz, jnp.concatenate(outs, axis=-1))


# speedup vs baseline: 1.2155x; 1.2155x over previous
import functools
import math

import jax
import jax.numpy as jnp
from jax import lax
from jax.experimental import pallas as pl
from jax.experimental.pallas import tpu as pltpu
from jax.experimental.pallas import tpu_sc as plsc

B = 4
N = 8192
NPOINT = 1024
NS_MAX = 32
EPS = 1e-5
BLK = 64


def _select_body(xyz_ref, c_ref, idx_ref):
    px = xyz_ref[0, 0, :][None, :]
    py = xyz_ref[0, 1, :][None, :]
    pz = xyz_ref[0, 2, :][None, :]
    cx = c_ref[0, :, 0:1]
    cy = c_ref[0, :, 1:2]
    cz = c_ref[0, :, 2:3]
    dx = cx - px
    dy = cy - py
    dz = cz - pz
    d2 = (dx * dx + dy * dy) + dz * dz
    lane = lax.broadcasted_iota(jnp.int32, (BLK, N), 1)
    big = jnp.float32(jnp.inf)
    idx_cols = []
    for _ in range(NS_MAX):
        jmin = jnp.argmin(d2, axis=1, keepdims=True)
        d2 = jnp.where(lane == jmin, big, d2)
        idx_cols.append(jmin.astype(jnp.int32))
    idx_ref[0] = jnp.concatenate(idx_cols, axis=1)


def _select(xyz_t, new_xyz):
    grid = (B, NPOINT // BLK)
    return pl.pallas_call(
        _select_body,
        grid=grid,
        in_specs=[
            pl.BlockSpec((1, 3, N), lambda b, i: (b, 0, 0)),
            pl.BlockSpec((1, BLK, 3), lambda b, i: (b, i, 0)),
        ],
        out_specs=pl.BlockSpec((1, BLK, NS_MAX), lambda b, i: (b, i, 0)),
        out_shape=jax.ShapeDtypeStruct((B, NPOINT, NS_MAX), jnp.int32),
    )(xyz_t, new_xyz)


_TOTAL = B * NPOINT * NS_MAX
_NW = 32
_CHUNK = _TOTAL // _NW


def _gather_body(idx_hbm, xyz_hbm, ox_hbm, oy_hbm, oz_hbm,
                 idx_v, xs_v, ys_v, zs_v, ox_v, oy_v, oz_v):
    wid = lax.axis_index("s") * 2 + lax.axis_index("c")
    batch = wid // (_NW // B)
    base = wid * _CHUNK
    pltpu.sync_copy(idx_hbm.at[pl.ds(base, _CHUNK)], idx_v)
    pltpu.sync_copy(xyz_hbm.at[batch * 3 + 0], xs_v)
    pltpu.sync_copy(xyz_hbm.at[batch * 3 + 1], ys_v)
    pltpu.sync_copy(xyz_hbm.at[batch * 3 + 2], zs_v)
    for i in range(_CHUNK // 16):
        iv = idx_v[pl.ds(i * 16, 16)]
        ox_v[pl.ds(i * 16, 16)] = plsc.load_gather(xs_v, [iv])
        oy_v[pl.ds(i * 16, 16)] = plsc.load_gather(ys_v, [iv])
        oz_v[pl.ds(i * 16, 16)] = plsc.load_gather(zs_v, [iv])
    pltpu.sync_copy(ox_v, ox_hbm.at[pl.ds(base, _CHUNK)])
    pltpu.sync_copy(oy_v, oy_hbm.at[pl.ds(base, _CHUNK)])
    pltpu.sync_copy(oz_v, oz_hbm.at[pl.ds(base, _CHUNK)])


def _gather(idx_flat, xyz_rows):
    mesh = plsc.VectorSubcoreMesh(core_axis_name="c", subcore_axis_name="s")
    k = functools.partial(
        pl.kernel,
        mesh=mesh,
        compiler_params=pltpu.CompilerParams(needs_layout_passes=False),
        out_type=[jax.ShapeDtypeStruct((_TOTAL,), jnp.float32)] * 3,
        scratch_types=[
            pltpu.VMEM((_CHUNK,), jnp.int32),
            pltpu.VMEM((N,), jnp.float32),
            pltpu.VMEM((N,), jnp.float32),
            pltpu.VMEM((N,), jnp.float32),
            pltpu.VMEM((_CHUNK,), jnp.float32),
            pltpu.VMEM((_CHUNK,), jnp.float32),
            pltpu.VMEM((_CHUNK,), jnp.float32),
        ],
    )(_gather_body)
    return k(idx_flat, xyz_rows)


def _branch_body(r2, inv_sqrt_d,
                 gx_ref, gy_ref, gz_ref, c_ref,
                 w1x_ref, w1y_ref, w1z_ref, b1_ref,
                 w2_ref, b2_ref, w3_ref, b3_ref,
                 wq_ref, wk_ref, wv_ref, out_ref):
    dot = functools.partial(
        lax.dot_general,
        preferred_element_type=jnp.float32,
    )
    mm = lambda a, b: dot(a, b, (((1,), (0,)), ((), ())))

    cx = c_ref[0, :, 0:1]
    cy = c_ref[0, :, 1:2]
    cz = c_ref[0, :, 2:3]
    relx = gx_ref[0] - cx
    rely = gy_ref[0] - cy
    relz = gz_ref[0] - cz
    d2 = (relx * relx + rely * rely) + relz * relz
    m = d2 <= r2
    relx = jnp.where(m, relx, jnp.broadcast_to(relx[:, 0:1], relx.shape))
    rely = jnp.where(m, rely, jnp.broadcast_to(rely[:, 0:1], rely.shape))
    relz = jnp.where(m, relz, jnp.broadcast_to(relz[:, 0:1], relz.shape))

    h = mm(relx, w1x_ref[...]) + mm(rely, w1y_ref[...]) + mm(relz, w1z_ref[...])
    h = jnp.maximum(h + b1_ref[...], 0.0)
    h = jnp.maximum(mm(h, w2_ref[...]) + b2_ref[...], 0.0)
    h = jnp.maximum(mm(h, w3_ref[...]) + b3_ref[...], 0.0)
    q = mm(h, wq_ref[...])
    k = mm(h, wk_ref[...])
    v = mm(h, wv_ref[...])
    s = dot(q, k, (((1,), (1,)), ((), ()))) * inv_sqrt_d
    smax = jnp.max(s, axis=1, keepdims=True)
    e = jnp.exp(s - smax)
    a = e / jnp.sum(e, axis=1, keepdims=True)
    out_ref[0] = mm(a, v)


def _fold_layer(layer):
    s = (layer["g1"] * layer["g2"]) / (1.0 + EPS)
    return layer["W"] * s[None, :], (layer["b"] * s)[None, :]


def _branch(gx, gy, gz, new_xyz, params, radius, ns):
    w1, b1 = _fold_layer(params["layers"][0])
    w2, b2 = _fold_layer(params["layers"][1])
    w3, b3 = _fold_layer(params["layers"][2])
    w1x, w1y, w1z = w1[0::3], w1[1::3], w1[2::3]
    body = functools.partial(_branch_body, radius * radius, 1.0 / math.sqrt(128.0))
    full = lambda a: pl.BlockSpec(a.shape, lambda b: (0,) * a.ndim)
    args = (gx, gy, gz, new_xyz,
            w1x, w1y, w1z, b1, w2, b2, w3, b3,
            params["Wq"], params["Wk"], params["Wv"])
    in_specs = [
        pl.BlockSpec((1, NPOINT, ns), lambda b: (b, 0, 0)),
        pl.BlockSpec((1, NPOINT, ns), lambda b: (b, 0, 0)),
        pl.BlockSpec((1, NPOINT, ns), lambda b: (b, 0, 0)),
        pl.BlockSpec((1, NPOINT, 3), lambda b: (b, 0, 0)),
    ] + [full(a) for a in args[4:]]
    return pl.pallas_call(
        body,
        grid=(B,),
        in_specs=in_specs,
        out_specs=pl.BlockSpec((1, NPOINT, 128), lambda b: (b, 0, 0)),
        out_shape=jax.ShapeDtypeStruct((B, NPOINT, 128), jnp.float32),
    )(*args)


def kernel(xyz, params0, params1):
    new_xyz = xyz[:, :NPOINT, :]
    xyz_t = jnp.transpose(xyz, (0, 2, 1))
    idx = _select(xyz_t, new_xyz)
    gx, gy, gz = _gather(idx.reshape(-1), xyz_t.reshape(B * 3, N))
    gx = gx.reshape(B, NPOINT, NS_MAX)
    gy = gy.reshape(B, NPOINT, NS_MAX)
    gz = gz.reshape(B, NPOINT, NS_MAX)
    outs = []
    for params, radius, ns in ((params0, 0.1, 16), (params1, 0.2, 32)):
        outs.append(_branch(gx[..., :ns], gy[..., :ns], gz[..., :ns],
                            new_xyz, params, radius, ns))
    return (new_xyz, jnp.concatenate(outs, axis=-1))


# speedup vs baseline: 1.2296x; 1.0116x over previous
import functools
import math

import jax
import jax.numpy as jnp
from jax import lax
from jax.experimental import pallas as pl
from jax.experimental.pallas import tpu as pltpu
from jax.experimental.pallas import tpu_sc as plsc

B = 4
N = 8192
NPOINT = 1024
NS_MAX = 32
EPS = 1e-5
BLK = 64


def _select_body(xyz_ref, c_ref, idx_ref):
    px = xyz_ref[0, 0, :][None, :]
    py = xyz_ref[0, 1, :][None, :]
    pz = xyz_ref[0, 2, :][None, :]
    cx = c_ref[0, :, 0:1]
    cy = c_ref[0, :, 1:2]
    cz = c_ref[0, :, 2:3]
    dx = cx - px
    dy = cy - py
    dz = cz - pz
    d2 = (dx * dx + dy * dy) + dz * dz
    lane = lax.broadcasted_iota(jnp.int32, (BLK, N), 1)
    big = jnp.float32(jnp.inf)
    idx_cols = []
    for _ in range(NS_MAX):
        jmin = jnp.argmin(d2, axis=1, keepdims=True)
        d2 = jnp.where(lane == jmin, big, d2)
        idx_cols.append(jmin.astype(jnp.int32))
    idx_ref[0] = jnp.concatenate(idx_cols, axis=1)


def _select(xyz_t, new_xyz):
    grid = (B, NPOINT // BLK)
    return pl.pallas_call(
        _select_body,
        grid=grid,
        in_specs=[
            pl.BlockSpec((1, 3, N), lambda b, i: (b, 0, 0)),
            pl.BlockSpec((1, BLK, 3), lambda b, i: (b, i, 0)),
        ],
        out_specs=pl.BlockSpec((1, BLK, NS_MAX), lambda b, i: (b, i, 0)),
        out_shape=jax.ShapeDtypeStruct((B, NPOINT, NS_MAX), jnp.int32),
    )(xyz_t, new_xyz)


_TOTAL = B * NPOINT * NS_MAX
_NW = 32
_CHUNK = _TOTAL // _NW


def _gather_body(idx_hbm, xyz_hbm, ox_hbm, oy_hbm, oz_hbm,
                 idx_v, xs_v, ys_v, zs_v, ox_v, oy_v, oz_v):
    wid = lax.axis_index("s") * 2 + lax.axis_index("c")
    batch = wid // (_NW // B)
    base = wid * _CHUNK
    pltpu.sync_copy(idx_hbm.at[pl.ds(base, _CHUNK)], idx_v)
    pltpu.sync_copy(xyz_hbm.at[batch * 3 + 0], xs_v)
    pltpu.sync_copy(xyz_hbm.at[batch * 3 + 1], ys_v)
    pltpu.sync_copy(xyz_hbm.at[batch * 3 + 2], zs_v)
    for i in range(_CHUNK // 16):
        iv = idx_v[pl.ds(i * 16, 16)]
        ox_v[pl.ds(i * 16, 16)] = plsc.load_gather(xs_v, [iv])
        oy_v[pl.ds(i * 16, 16)] = plsc.load_gather(ys_v, [iv])
        oz_v[pl.ds(i * 16, 16)] = plsc.load_gather(zs_v, [iv])
    pltpu.sync_copy(ox_v, ox_hbm.at[pl.ds(base, _CHUNK)])
    pltpu.sync_copy(oy_v, oy_hbm.at[pl.ds(base, _CHUNK)])
    pltpu.sync_copy(oz_v, oz_hbm.at[pl.ds(base, _CHUNK)])


def _gather(idx_flat, xyz_rows):
    mesh = plsc.VectorSubcoreMesh(core_axis_name="c", subcore_axis_name="s")
    k = functools.partial(
        pl.kernel,
        mesh=mesh,
        compiler_params=pltpu.CompilerParams(needs_layout_passes=False),
        out_type=[jax.ShapeDtypeStruct((_TOTAL,), jnp.float32)] * 3,
        scratch_types=[
            pltpu.VMEM((_CHUNK,), jnp.int32),
            pltpu.VMEM((N,), jnp.float32),
            pltpu.VMEM((N,), jnp.float32),
            pltpu.VMEM((N,), jnp.float32),
            pltpu.VMEM((_CHUNK,), jnp.float32),
            pltpu.VMEM((_CHUNK,), jnp.float32),
            pltpu.VMEM((_CHUNK,), jnp.float32),
        ],
    )(_gather_body)
    return k(idx_flat, xyz_rows)


_RADII = (0.1, 0.2)
_NSAMPLES = (16, 32)


def _branches_body(gx_ref, gy_ref, gz_ref, c_ref,
                   w1x0_ref, w1y0_ref, w1z0_ref, b10_ref,
                   w20_ref, b20_ref, w30_ref, b30_ref,
                   wq0_ref, wk0_ref, wv0_ref,
                   w1x1_ref, w1y1_ref, w1z1_ref, b11_ref,
                   w21_ref, b21_ref, w31_ref, b31_ref,
                   wq1_ref, wk1_ref, wv1_ref, out_ref):
    dot = functools.partial(
        lax.dot_general,
        preferred_element_type=jnp.float32,
    )
    mm = lambda a, b: dot(a, b, (((1,), (0,)), ((), ())))
    inv_sqrt_d = 1.0 / math.sqrt(128.0)

    cx = c_ref[0, :, 0:1]
    cy = c_ref[0, :, 1:2]
    cz = c_ref[0, :, 2:3]
    relx = gx_ref[0] - cx
    rely = gy_ref[0] - cy
    relz = gz_ref[0] - cz
    d2 = (relx * relx + rely * rely) + relz * relz

    weights = (
        (w1x0_ref, w1y0_ref, w1z0_ref, b10_ref, w20_ref, b20_ref,
         w30_ref, b30_ref, wq0_ref, wk0_ref, wv0_ref),
        (w1x1_ref, w1y1_ref, w1z1_ref, b11_ref, w21_ref, b21_ref,
         w31_ref, b31_ref, wq1_ref, wk1_ref, wv1_ref),
    )
    for bi, (radius, ns) in enumerate(zip(_RADII, _NSAMPLES)):
        m = d2[:, :ns] <= radius * radius
        rx = relx[:, :ns]
        ry = rely[:, :ns]
        rz = relz[:, :ns]
        rx = jnp.where(m, rx, jnp.broadcast_to(rx[:, 0:1], rx.shape))
        ry = jnp.where(m, ry, jnp.broadcast_to(ry[:, 0:1], ry.shape))
        rz = jnp.where(m, rz, jnp.broadcast_to(rz[:, 0:1], rz.shape))
        (w1x, w1y, w1z, b1, w2, b2, w3, b3, wq, wk, wv) = weights[bi]
        h = mm(rx, w1x[...]) + mm(ry, w1y[...]) + mm(rz, w1z[...])
        h = jnp.maximum(h + b1[...], 0.0)
        h = jnp.maximum(mm(h, w2[...]) + b2[...], 0.0)
        h = jnp.maximum(mm(h, w3[...]) + b3[...], 0.0)
        q = mm(h, wq[...])
        k = mm(h, wk[...])
        v = mm(h, wv[...])
        s = dot(q, k, (((1,), (1,)), ((), ()))) * inv_sqrt_d
        smax = jnp.max(s, axis=1, keepdims=True)
        e = jnp.exp(s - smax)
        a = e / jnp.sum(e, axis=1, keepdims=True)
        out_ref[0, :, bi * 128:(bi + 1) * 128] = mm(a, v)


def _fold_layer(layer):
    s = (layer["g1"] * layer["g2"]) / (1.0 + EPS)
    return layer["W"] * s[None, :], (layer["b"] * s)[None, :]


def _branch_weights(params):
    w1, b1 = _fold_layer(params["layers"][0])
    w2, b2 = _fold_layer(params["layers"][1])
    w3, b3 = _fold_layer(params["layers"][2])
    w1x, w1y, w1z = w1[0::3], w1[1::3], w1[2::3]
    return (w1x, w1y, w1z, b1, w2, b2, w3, b3,
            params["Wq"], params["Wk"], params["Wv"])


def _branches(gx, gy, gz, new_xyz, params0, params1):
    full = lambda a: pl.BlockSpec(a.shape, lambda b: (0,) * a.ndim)
    args = ((gx, gy, gz, new_xyz)
            + _branch_weights(params0) + _branch_weights(params1))
    in_specs = [
        pl.BlockSpec((1, NPOINT, NS_MAX), lambda b: (b, 0, 0)),
        pl.BlockSpec((1, NPOINT, NS_MAX), lambda b: (b, 0, 0)),
        pl.BlockSpec((1, NPOINT, NS_MAX), lambda b: (b, 0, 0)),
        pl.BlockSpec((1, NPOINT, 3), lambda b: (b, 0, 0)),
    ] + [full(a) for a in args[4:]]
    return pl.pallas_call(
        _branches_body,
        grid=(B,),
        in_specs=in_specs,
        out_specs=pl.BlockSpec((1, NPOINT, 256), lambda b: (b, 0, 0)),
        out_shape=jax.ShapeDtypeStruct((B, NPOINT, 256), jnp.float32),
    )(*args)


def kernel(xyz, params0, params1):
    new_xyz = xyz[:, :NPOINT, :]
    xyz_t = jnp.transpose(xyz, (0, 2, 1))
    idx = _select(xyz_t, new_xyz)
    gx, gy, gz = _gather(idx.reshape(-1), xyz_t.reshape(B * 3, N))
    gx = gx.reshape(B, NPOINT, NS_MAX)
    gy = gy.reshape(B, NPOINT, NS_MAX)
    gz = gz.reshape(B, NPOINT, NS_MAX)
    out = _branches(gx, gy, gz, new_xyz, params0, params1)
    return (new_xyz, out)
